# baseline (speedup 1.0000x reference)
import functools

import jax
import jax.numpy as jnp
from jax import lax
from jax.experimental import pallas as pl
from jax.experimental.pallas import tpu as pltpu
from jax.experimental.pallas import tpu_sc as plsc

B = 16384
D = 100
BLK = 2 * D
CHUNK = 128
LANES = 16


def _tc_gather_call():
    grid_spec = pltpu.PrefetchScalarGridSpec(
        num_scalar_prefetch=1,
        grid=(1,),
        in_specs=[pl.BlockSpec(memory_space=pl.MemorySpace.ANY)],
        out_specs=pl.BlockSpec(memory_space=pl.MemorySpace.ANY),
        scratch_shapes=[pltpu.SemaphoreType.DMA],
    )

    def body(idx_ref, ut_ref, out_ref, sem):
        def step(i, _):
            r = idx_ref[i]
            pltpu.make_async_copy(
                ut_ref.at[pl.ds(r, 1)], out_ref.at[pl.ds(i, 1)], sem
            ).start()
            return 0

        lax.fori_loop(0, B, step, 0, unroll=8)
        pltpu.make_async_copy(
            ut_ref.at[pl.ds(0, B)], out_ref, sem
        ).wait()

    return pl.pallas_call(
        body,
        grid_spec=grid_spec,
        out_shape=jax.ShapeDtypeStruct((B, D), jnp.float32),
    )


def _sc_score_call():
    info = plsc.get_sparse_core_info()
    nc, ns = info.num_cores, info.num_subcores
    nw = nc * ns
    b_per_w = B // nw
    n_chunks = b_per_w // CHUNK
    mesh = plsc.VectorSubcoreMesh(core_axis_name="c", subcore_axis_name="s")

    @functools.partial(
        pl.kernel,
        out_type=(
            jax.ShapeDtypeStruct((B,), jnp.float32),
            jax.ShapeDtypeStruct((B,), jnp.float32),
        ),
        mesh=mesh,
        compiler_params=pltpu.CompilerParams(use_tc_tiling_on_sc=False,
                                             needs_layout_passes=False),
        scratch_types=[
            pltpu.VMEM((b_per_w * D,), jnp.float32),
            pltpu.VMEM((CHUNK,), jnp.int32),
            pltpu.VMEM((CHUNK,), jnp.int32),
            pltpu.VMEM((CHUNK,), jnp.int32),
            pltpu.VMEM((CHUNK,), jnp.int32),
            pltpu.VMEM((CHUNK, BLK), jnp.float32),
            pltpu.VMEM((CHUNK, BLK), jnp.float32),
            pltpu.VMEM((CHUNK,), jnp.float32),
            pltpu.VMEM((CHUNK,), jnp.float32),
            pltpu.SemaphoreType.DMA,
            pltpu.SemaphoreType.DMA,
        ],
    )
    def sc_call(pb_hbm, nb_hbm, po_hbm, no_hbm, it_hbm, uf_hbm,
                pos_hbm, neg_hbm,
                u_loc, idx_p, idx_n, off_p, off_n, p_rows, n_rows,
                pos_c, neg_c, sem_u, sem):
        wid = lax.axis_index("s") * nc + lax.axis_index("c")
        lane = lax.iota(jnp.int32, LANES)
        zeros = jnp.zeros((LANES,), jnp.float32)

        cu = pltpu.async_copy(
            uf_hbm.at[pl.ds(wid * b_per_w * D, b_per_w * D)], u_loc, sem_u)

        for c in range(n_chunks):
            base = wid * b_per_w + c * CHUNK
            pltpu.sync_copy(pb_hbm.at[pl.ds(base, CHUNK)], idx_p)
            pltpu.sync_copy(nb_hbm.at[pl.ds(base, CHUNK)], idx_n)
            pltpu.sync_copy(po_hbm.at[pl.ds(base, CHUNK)], off_p)
            pltpu.sync_copy(no_hbm.at[pl.ds(base, CHUNK)], off_n)
            cp = pltpu.async_copy(it_hbm.at[idx_p], p_rows, sem)
            cn = pltpu.async_copy(it_hbm.at[idx_n], n_rows, sem)
            cp.wait()
            cn.wait()
            if c == 0:
                cu.wait()

            def group_body(g, _):
                rows = g * LANES + lane
                ov_p = off_p[pl.ds(g * LANES, LANES)]
                ov_n = off_n[pl.ds(g * LANES, LANES)]
                u_idx0 = (c * CHUNK + rows) * D

                def d_step(d, carry):
                    acc_p, acc_n, ui_, cp_, cn_ = carry
                    u = plsc.load_gather(u_loc, [ui_])
                    p = plsc.load_gather(p_rows, [rows, cp_])
                    n = plsc.load_gather(n_rows, [rows, cn_])
                    return (acc_p + u * p, acc_n + u * n,
                            ui_ + 1, cp_ + 1, cn_ + 1)

                acc_p, acc_n, _, _, _ = lax.fori_loop(
                    0, D, d_step, (zeros, zeros, u_idx0, ov_p, ov_n),
                    unroll=4)
                pos_c[pl.ds(g * LANES, LANES)] = acc_p
                neg_c[pl.ds(g * LANES, LANES)] = acc_n
                return 0

            lax.fori_loop(0, CHUNK // LANES, group_body, 0)
            pltpu.sync_copy(pos_c, pos_hbm.at[pl.ds(base, CHUNK)])
            pltpu.sync_copy(neg_c, neg_hbm.at[pl.ds(base, CHUNK)])

    return sc_call


def kernel(user_inputs, pos_inputs, neg_inputs, user_table, item_table):
    ui = jnp.squeeze(user_inputs, axis=-1)
    pi = jnp.squeeze(pos_inputs, axis=-1)
    ni = jnp.squeeze(neg_inputs, axis=-1)
    u_rows = _tc_gather_call()(ui, user_table)
    u_flat = u_rows.reshape(-1)
    it2 = item_table.reshape(item_table.shape[0] // 2, BLK)
    pos, neg = _sc_score_call()(
        pi >> 1, ni >> 1, (pi & 1) * D, (ni & 1) * D, it2, u_flat)
    return (pos[:, None], neg[:, None])


# speedup vs baseline: 1.0691x; 1.0691x over previous
import functools

import jax
import jax.numpy as jnp
from jax import lax
from jax.experimental import pallas as pl
from jax.experimental.pallas import tpu as pltpu
from jax.experimental.pallas import tpu_sc as plsc

B = 16384
D = 100
BLK = 2 * D
CHUNK = 128
LANES = 16


def _tc_gather_call():
    half = B // 2
    grid_spec = pltpu.PrefetchScalarGridSpec(
        num_scalar_prefetch=1,
        grid=(1,),
        in_specs=[pl.BlockSpec(memory_space=pl.MemorySpace.ANY)],
        out_specs=pl.BlockSpec(memory_space=pl.MemorySpace.ANY),
        scratch_shapes=[
            pltpu.VMEM((half, D), jnp.float32),
            pltpu.SemaphoreType.DMA,
            pltpu.SemaphoreType.DMA,
        ],
    )

    def body(idx_ref, ut_ref, out_ref, vbuf, sem_h, sem_v):
        def step(i, _):
            pltpu.make_async_copy(
                ut_ref.at[pl.ds(idx_ref[i], 1)], out_ref.at[pl.ds(i, 1)],
                sem_h,
            ).start()
            j = i + half
            pltpu.make_async_copy(
                ut_ref.at[pl.ds(idx_ref[j], 1)], vbuf.at[pl.ds(i, 1)],
                sem_v,
            ).start()
            return 0

        lax.fori_loop(0, half, step, 0, unroll=8)
        pltpu.make_async_copy(
            ut_ref.at[pl.ds(0, half)], vbuf, sem_v
        ).wait()
        pltpu.sync_copy(vbuf, out_ref.at[pl.ds(half, half)])
        pltpu.make_async_copy(
            ut_ref.at[pl.ds(0, half)], out_ref.at[pl.ds(0, half)], sem_h
        ).wait()

    return pl.pallas_call(
        body,
        grid_spec=grid_spec,
        out_shape=jax.ShapeDtypeStruct((B, D), jnp.float32),
    )


def _sc_score_call():
    info = plsc.get_sparse_core_info()
    nc, ns = info.num_cores, info.num_subcores
    nw = nc * ns
    b_per_w = B // nw
    n_chunks = b_per_w // CHUNK
    mesh = plsc.VectorSubcoreMesh(core_axis_name="c", subcore_axis_name="s")

    @functools.partial(
        pl.kernel,
        out_type=(
            jax.ShapeDtypeStruct((B,), jnp.float32),
            jax.ShapeDtypeStruct((B,), jnp.float32),
        ),
        mesh=mesh,
        compiler_params=pltpu.CompilerParams(use_tc_tiling_on_sc=False,
                                             needs_layout_passes=False),
        scratch_types=[
            pltpu.VMEM((b_per_w * D,), jnp.float32),
            pltpu.VMEM((CHUNK,), jnp.int32),
            pltpu.VMEM((CHUNK,), jnp.int32),
            pltpu.VMEM((CHUNK,), jnp.int32),
            pltpu.VMEM((CHUNK,), jnp.int32),
            pltpu.VMEM((CHUNK, BLK), jnp.float32),
            pltpu.VMEM((CHUNK, BLK), jnp.float32),
            pltpu.VMEM((CHUNK,), jnp.float32),
            pltpu.VMEM((CHUNK,), jnp.float32),
            pltpu.SemaphoreType.DMA,
            pltpu.SemaphoreType.DMA,
        ],
    )
    def sc_call(pb_hbm, nb_hbm, po_hbm, no_hbm, it_hbm, uf_hbm,
                pos_hbm, neg_hbm,
                u_loc, idx_p, idx_n, off_p, off_n, p_rows, n_rows,
                pos_c, neg_c, sem_u, sem):
        wid = lax.axis_index("s") * nc + lax.axis_index("c")
        lane = lax.iota(jnp.int32, LANES)
        zeros = jnp.zeros((LANES,), jnp.float32)

        cu = pltpu.async_copy(
            uf_hbm.at[pl.ds(wid * b_per_w * D, b_per_w * D)], u_loc, sem_u)

        for c in range(n_chunks):
            base = wid * b_per_w + c * CHUNK
            pltpu.sync_copy(pb_hbm.at[pl.ds(base, CHUNK)], idx_p)
            pltpu.sync_copy(nb_hbm.at[pl.ds(base, CHUNK)], idx_n)
            pltpu.sync_copy(po_hbm.at[pl.ds(base, CHUNK)], off_p)
            pltpu.sync_copy(no_hbm.at[pl.ds(base, CHUNK)], off_n)
            cp = pltpu.async_copy(it_hbm.at[idx_p], p_rows, sem)
            cn = pltpu.async_copy(it_hbm.at[idx_n], n_rows, sem)
            cp.wait()
            cn.wait()
            if c == 0:
                cu.wait()

            def group_body(g, _):
                rows = g * LANES + lane
                ov_p = off_p[pl.ds(g * LANES, LANES)]
                ov_n = off_n[pl.ds(g * LANES, LANES)]
                u_idx0 = (c * CHUNK + rows) * D

                def d_step(d, carry):
                    acc_p, acc_n, ui_, cp_, cn_ = carry
                    u = plsc.load_gather(u_loc, [ui_])
                    p = plsc.load_gather(p_rows, [rows, cp_])
                    n = plsc.load_gather(n_rows, [rows, cn_])
                    return (acc_p + u * p, acc_n + u * n,
                            ui_ + 1, cp_ + 1, cn_ + 1)

                acc_p, acc_n, _, _, _ = lax.fori_loop(
                    0, D, d_step, (zeros, zeros, u_idx0, ov_p, ov_n),
                    unroll=4)
                pos_c[pl.ds(g * LANES, LANES)] = acc_p
                neg_c[pl.ds(g * LANES, LANES)] = acc_n
                return 0

            lax.fori_loop(0, CHUNK // LANES, group_body, 0)
            pltpu.sync_copy(pos_c, pos_hbm.at[pl.ds(base, CHUNK)])
            pltpu.sync_copy(neg_c, neg_hbm.at[pl.ds(base, CHUNK)])

    return sc_call


def kernel(user_inputs, pos_inputs, neg_inputs, user_table, item_table):
    ui = jnp.squeeze(user_inputs, axis=-1)
    pi = jnp.squeeze(pos_inputs, axis=-1)
    ni = jnp.squeeze(neg_inputs, axis=-1)
    u_rows = _tc_gather_call()(ui, user_table)
    u_flat = u_rows.reshape(-1)
    it2 = item_table.reshape(item_table.shape[0] // 2, BLK)
    pos, neg = _sc_score_call()(
        pi >> 1, ni >> 1, (pi & 1) * D, (ni & 1) * D, it2, u_flat)
    return (pos[:, None], neg[:, None])


# speedup vs baseline: 1.1032x; 1.0320x over previous
import functools

import jax
import jax.numpy as jnp
from jax import lax
from jax.experimental import pallas as pl
from jax.experimental.pallas import tpu as pltpu
from jax.experimental.pallas import tpu_sc as plsc

B = 16384
D = 100
BLK = 2 * D
CHUNK = 128
LANES = 16


def _tc_gather_call():
    half = B // 2
    grid_spec = pltpu.PrefetchScalarGridSpec(
        num_scalar_prefetch=1,
        grid=(1,),
        in_specs=[pl.BlockSpec(memory_space=pl.MemorySpace.ANY)],
        out_specs=pl.BlockSpec(memory_space=pl.MemorySpace.ANY),
        scratch_shapes=[
            pltpu.VMEM((B, D), jnp.float32),
            pltpu.SemaphoreType.DMA,
            pltpu.SemaphoreType.DMA,
        ],
    )

    def body(idx_ref, ut_ref, out_ref, vbuf, sem_h, sem_v):
        def step(i, _):
            pltpu.make_async_copy(
                ut_ref.at[pl.ds(idx_ref[i], 1)], vbuf.at[pl.ds(i, 1)],
                sem_h,
            ).start()
            j = i + half
            pltpu.make_async_copy(
                ut_ref.at[pl.ds(idx_ref[j], 1)], vbuf.at[pl.ds(j, 1)],
                sem_v,
            ).start()
            return 0

        lax.fori_loop(0, half, step, 0, unroll=8)
        pltpu.make_async_copy(
            ut_ref.at[pl.ds(0, half)], vbuf.at[pl.ds(0, half)], sem_h
        ).wait()
        pltpu.sync_copy(vbuf.at[pl.ds(0, half)], out_ref.at[pl.ds(0, half)])
        pltpu.make_async_copy(
            ut_ref.at[pl.ds(0, half)], vbuf.at[pl.ds(0, half)], sem_v
        ).wait()
        pltpu.sync_copy(vbuf.at[pl.ds(half, half)],
                        out_ref.at[pl.ds(half, half)])

    return pl.pallas_call(
        body,
        grid_spec=grid_spec,
        out_shape=jax.ShapeDtypeStruct((B, D), jnp.float32),
    )


def _sc_score_call():
    info = plsc.get_sparse_core_info()
    nc, ns = info.num_cores, info.num_subcores
    nw = nc * ns
    b_per_w = B // nw
    n_chunks = b_per_w // CHUNK
    mesh = plsc.VectorSubcoreMesh(core_axis_name="c", subcore_axis_name="s")

    @functools.partial(
        pl.kernel,
        out_type=(
            jax.ShapeDtypeStruct((B,), jnp.float32),
            jax.ShapeDtypeStruct((B,), jnp.float32),
        ),
        mesh=mesh,
        compiler_params=pltpu.CompilerParams(use_tc_tiling_on_sc=False,
                                             needs_layout_passes=False),
        scratch_types=[
            pltpu.VMEM((b_per_w * D,), jnp.float32),
            pltpu.VMEM((CHUNK,), jnp.int32),
            pltpu.VMEM((CHUNK,), jnp.int32),
            pltpu.VMEM((CHUNK,), jnp.int32),
            pltpu.VMEM((CHUNK,), jnp.int32),
            pltpu.VMEM((CHUNK, BLK), jnp.float32),
            pltpu.VMEM((CHUNK, BLK), jnp.float32),
            pltpu.VMEM((CHUNK,), jnp.float32),
            pltpu.VMEM((CHUNK,), jnp.float32),
            pltpu.SemaphoreType.DMA,
            pltpu.SemaphoreType.DMA,
        ],
    )
    def sc_call(pb_hbm, nb_hbm, po_hbm, no_hbm, it_hbm, uf_hbm,
                pos_hbm, neg_hbm,
                u_loc, idx_p, idx_n, off_p, off_n, p_rows, n_rows,
                pos_c, neg_c, sem_u, sem):
        wid = lax.axis_index("s") * nc + lax.axis_index("c")
        lane = lax.iota(jnp.int32, LANES)
        zeros = jnp.zeros((LANES,), jnp.float32)

        cu = pltpu.async_copy(
            uf_hbm.at[pl.ds(wid * b_per_w * D, b_per_w * D)], u_loc, sem_u)

        for c in range(n_chunks):
            base = wid * b_per_w + c * CHUNK
            pltpu.sync_copy(pb_hbm.at[pl.ds(base, CHUNK)], idx_p)
            pltpu.sync_copy(nb_hbm.at[pl.ds(base, CHUNK)], idx_n)
            pltpu.sync_copy(po_hbm.at[pl.ds(base, CHUNK)], off_p)
            pltpu.sync_copy(no_hbm.at[pl.ds(base, CHUNK)], off_n)
            cp = pltpu.async_copy(it_hbm.at[idx_p], p_rows, sem)
            cn = pltpu.async_copy(it_hbm.at[idx_n], n_rows, sem)
            cp.wait()
            cn.wait()
            if c == 0:
                cu.wait()

            def group_body(g, _):
                rows = g * LANES + lane
                ov_p = off_p[pl.ds(g * LANES, LANES)]
                ov_n = off_n[pl.ds(g * LANES, LANES)]
                u_idx0 = (c * CHUNK + rows) * D

                def d_step(d, carry):
                    acc_p, acc_n, ui_, cp_, cn_ = carry
                    u = plsc.load_gather(u_loc, [ui_])
                    p = plsc.load_gather(p_rows, [rows, cp_])
                    n = plsc.load_gather(n_rows, [rows, cn_])
                    return (acc_p + u * p, acc_n + u * n,
                            ui_ + 1, cp_ + 1, cn_ + 1)

                acc_p, acc_n, _, _, _ = lax.fori_loop(
                    0, D, d_step, (zeros, zeros, u_idx0, ov_p, ov_n),
                    unroll=4)
                pos_c[pl.ds(g * LANES, LANES)] = acc_p
                neg_c[pl.ds(g * LANES, LANES)] = acc_n
                return 0

            lax.fori_loop(0, CHUNK // LANES, group_body, 0)
            pltpu.sync_copy(pos_c, pos_hbm.at[pl.ds(base, CHUNK)])
            pltpu.sync_copy(neg_c, neg_hbm.at[pl.ds(base, CHUNK)])

    return sc_call


def kernel(user_inputs, pos_inputs, neg_inputs, user_table, item_table):
    ui = jnp.squeeze(user_inputs, axis=-1)
    pi = jnp.squeeze(pos_inputs, axis=-1)
    ni = jnp.squeeze(neg_inputs, axis=-1)
    u_rows = _tc_gather_call()(ui, user_table)
    u_flat = u_rows.reshape(-1)
    it2 = item_table.reshape(item_table.shape[0] // 2, BLK)
    pos, neg = _sc_score_call()(
        pi >> 1, ni >> 1, (pi & 1) * D, (ni & 1) * D, it2, u_flat)
    return (pos[:, None], neg[:, None])
